# Initial kernel scaffold; baseline (speedup 1.0000x reference)
import jax
import jax.numpy as jnp
from jax.experimental import pallas as pl


def kernel(x, W, w1, b1, w2, b2):
    raise NotImplementedError("write your pallas kernel here")


# speedup vs baseline: 1.7862x; 1.7862x over previous
import jax
import jax.numpy as jnp
from jax.experimental import pallas as pl
from jax.experimental.pallas import tpu as pltpu

_N_TILE = 512
_MAX_K = 3


def _router_body(x_ref, wg_ref, w1_ref, b1_ref, w2t_ref, b2_ref,
                 wts_ref, idx_ref, mask_ref, k_ref):
    x = x_ref[...]

    h = jnp.dot(x, w1_ref[...], preferred_element_type=jnp.float32)
    h = jnp.maximum(h + b1_ref[...], 0.0)
    z = jnp.dot(h.astype(jnp.bfloat16), w2t_ref[...],
                preferred_element_type=jnp.float32)[:, 0:1] + b2_ref[...]
    score = jax.nn.sigmoid(z)
    kf = jnp.clip(jnp.round(score * float(_MAX_K)) + 1.0, 1.0, float(_MAX_K))
    k = kf.astype(jnp.int32)

    g = jnp.dot(x, wg_ref[...], preferred_element_type=jnp.float32)[:, :16]
    p = jax.nn.softmax(g, axis=-1)

    lane = jax.lax.broadcasted_iota(jnp.int32, p.shape, 1)
    work = p
    tvals = jnp.zeros_like(p)
    tinds = jnp.zeros_like(lane)
    mask = jnp.zeros_like(p)
    for j in range(_MAX_K):
        m = jnp.max(work, axis=-1, keepdims=True)
        i_j = jnp.min(jnp.where(work == m, lane, 16), axis=-1, keepdims=True)
        tvals = jnp.where(lane == j, m, tvals)
        tinds = jnp.where(lane == j, i_j, tinds)
        active_j = jnp.where((m > 0.0) & (j < k), 1.0, 0.0)
        mask = mask + jnp.where(lane == i_j, active_j, 0.0)
        work = jnp.where(lane == i_j, -jnp.inf, work)

    tp = tvals * (lane < k).astype(jnp.float32)
    w = tp / (jnp.sum(tp, axis=-1, keepdims=True) + 1e-8)

    wts_ref[...] = w
    idx_ref[...] = tinds
    mask_ref[...] = mask
    k_ref[...] = jnp.broadcast_to(k, k_ref.shape)


def kernel(x, W, w1, b1, w2, b2):
    B, S, D = x.shape
    E = W.shape[1]
    N = B * S
    x_flat = x.reshape(N, D).astype(jnp.bfloat16)
    Wp = jnp.pad(W, ((0, 0), (0, 128 - E))).astype(jnp.bfloat16)
    w1b = w1.astype(jnp.bfloat16)
    b1r = b1.reshape(1, -1)
    w2t = jnp.pad(w2, ((0, 0), (0, 127))).astype(jnp.bfloat16)
    b2r = b2.reshape(1, 1)

    grid = (N // _N_TILE,)
    T = _N_TILE
    wts, idx, mask, kout = pl.pallas_call(
        _router_body,
        grid=grid,
        in_specs=[
            pl.BlockSpec((T, D), lambda i: (i, 0)),
            pl.BlockSpec((D, 128), lambda i: (0, 0)),
            pl.BlockSpec((D, D // 2), lambda i: (0, 0)),
            pl.BlockSpec((1, D // 2), lambda i: (0, 0)),
            pl.BlockSpec((D // 2, 128), lambda i: (0, 0)),
            pl.BlockSpec((1, 1), lambda i: (0, 0)),
        ],
        out_specs=[
            pl.BlockSpec((T, E), lambda i: (i, 0)),
            pl.BlockSpec((T, E), lambda i: (i, 0)),
            pl.BlockSpec((T, E), lambda i: (i, 0)),
            pl.BlockSpec((T, E), lambda i: (i, 0)),
        ],
        out_shape=[
            jax.ShapeDtypeStruct((N, E), jnp.float32),
            jax.ShapeDtypeStruct((N, E), jnp.int32),
            jax.ShapeDtypeStruct((N, E), jnp.float32),
            jax.ShapeDtypeStruct((N, E), jnp.int32),
        ],
        compiler_params=pltpu.CompilerParams(
            dimension_semantics=("arbitrary",),
        ),
    )(x_flat, Wp, w1b, b1r, w2t, b2r)

    return (wts[:, :_MAX_K], idx[:, :_MAX_K],
            mask.reshape(B, S, E), kout[:, 0])


# speedup vs baseline: 1.8230x; 1.0206x over previous
import jax
import jax.numpy as jnp
from jax.experimental import pallas as pl
from jax.experimental.pallas import tpu as pltpu

_N_TILE = 1024
_MAX_K = 3
_H = 1024


def _router_body(x_ref, wcat_ref, b1_ref, w2t_ref, b2_ref,
                 wts_ref, idx_ref, mask_ref, k_ref):
    x = x_ref[...]

    hg = jnp.dot(x, wcat_ref[...], preferred_element_type=jnp.float32)
    h = jnp.maximum(hg[:, :_H] + b1_ref[...], 0.0)
    g = hg[:, _H:_H + 16]

    z = jnp.dot(h.astype(jnp.bfloat16), w2t_ref[...],
                preferred_element_type=jnp.float32)[:, 0:1] + b2_ref[...]
    score = jax.nn.sigmoid(z)
    kf = jnp.clip(jnp.round(score * float(_MAX_K)) + 1.0, 1.0, float(_MAX_K))
    k = kf.astype(jnp.int32)

    p = jax.nn.softmax(g, axis=-1)

    lane = jax.lax.broadcasted_iota(jnp.int32, p.shape, 1)
    work = p
    tvals = jnp.zeros_like(p)
    tinds = jnp.zeros_like(lane)
    mask = jnp.zeros_like(p)
    for j in range(_MAX_K):
        m = jnp.max(work, axis=-1, keepdims=True)
        i_j = jnp.min(jnp.where(work == m, lane, 16), axis=-1, keepdims=True)
        tvals = jnp.where(lane == j, m, tvals)
        tinds = jnp.where(lane == j, i_j, tinds)
        active_j = jnp.where((m > 0.0) & (j < k), 1.0, 0.0)
        mask = mask + jnp.where(lane == i_j, active_j, 0.0)
        work = jnp.where(lane == i_j, -jnp.inf, work)

    tp = tvals * (lane < k).astype(jnp.float32)
    w = tp / (jnp.sum(tp, axis=-1, keepdims=True) + 1e-8)

    wts_ref[...] = w
    idx_ref[...] = tinds
    mask_ref[...] = mask
    k_ref[...] = jnp.broadcast_to(k, k_ref.shape)


def kernel(x, W, w1, b1, w2, b2):
    B, S, D = x.shape
    E = W.shape[1]
    N = B * S
    x_flat = x.reshape(N, D).astype(jnp.bfloat16)
    wcat = jnp.concatenate(
        [w1, jnp.pad(W, ((0, 0), (0, 128 - E)))], axis=1
    ).astype(jnp.bfloat16)
    b1r = b1.reshape(1, -1)
    w2t = jnp.pad(w2, ((0, 0), (0, 127))).astype(jnp.bfloat16)
    b2r = b2.reshape(1, 1)

    grid = (N // _N_TILE,)
    T = _N_TILE
    wts, idx, mask, kout = pl.pallas_call(
        _router_body,
        grid=grid,
        in_specs=[
            pl.BlockSpec((T, D), lambda i: (i, 0)),
            pl.BlockSpec((D, _H + 128), lambda i: (0, 0)),
            pl.BlockSpec((1, _H), lambda i: (0, 0)),
            pl.BlockSpec((_H, 128), lambda i: (0, 0)),
            pl.BlockSpec((1, 1), lambda i: (0, 0)),
        ],
        out_specs=[
            pl.BlockSpec((T, E), lambda i: (i, 0)),
            pl.BlockSpec((T, E), lambda i: (i, 0)),
            pl.BlockSpec((T, E), lambda i: (i, 0)),
            pl.BlockSpec((T, E), lambda i: (i, 0)),
        ],
        out_shape=[
            jax.ShapeDtypeStruct((N, E), jnp.float32),
            jax.ShapeDtypeStruct((N, E), jnp.int32),
            jax.ShapeDtypeStruct((N, E), jnp.float32),
            jax.ShapeDtypeStruct((N, E), jnp.int32),
        ],
        compiler_params=pltpu.CompilerParams(
            dimension_semantics=("arbitrary",),
        ),
    )(x_flat, wcat, b1r, w2t, b2r)

    return (wts[:, :_MAX_K], idx[:, :_MAX_K],
            mask.reshape(B, S, E), kout[:, 0])


# speedup vs baseline: 2.2973x; 1.2602x over previous
import jax
import jax.numpy as jnp
from jax.experimental import pallas as pl
from jax.experimental.pallas import tpu as pltpu

_N_TILE = 1024
_MAX_K = 3
_H = 1024


def _router_body(x_ref, wcat_ref, b1_ref, w2t_ref, b2_ref,
                 wts_ref, idx_ref, mask_ref, k_ref):
    x = x_ref[...].astype(jnp.bfloat16)

    hg = jnp.dot(x, wcat_ref[...], preferred_element_type=jnp.float32)
    h = jnp.maximum(hg[:, :_H] + b1_ref[...], 0.0)
    g = hg[:, _H:_H + 16]

    z = jnp.dot(h.astype(jnp.bfloat16), w2t_ref[...],
                preferred_element_type=jnp.float32)[:, 0:1] + b2_ref[...]
    score = jax.nn.sigmoid(z)
    kf = jnp.clip(jnp.round(score * float(_MAX_K)) + 1.0, 1.0, float(_MAX_K))
    k = kf.astype(jnp.int32)

    p = jax.nn.softmax(g, axis=-1)

    lane = jax.lax.broadcasted_iota(jnp.int32, p.shape, 1)
    work = p
    tvals = jnp.zeros_like(p)
    tinds = jnp.zeros_like(lane)
    mask = jnp.zeros_like(p)
    for j in range(_MAX_K):
        m = jnp.max(work, axis=-1, keepdims=True)
        i_j = jnp.min(jnp.where(work == m, lane, 16), axis=-1, keepdims=True)
        tvals = jnp.where(lane == j, m, tvals)
        tinds = jnp.where(lane == j, i_j, tinds)
        active_j = jnp.where((m > 0.0) & (j < k), 1.0, 0.0)
        mask = mask + jnp.where(lane == i_j, active_j, 0.0)
        work = jnp.where(lane == i_j, -jnp.inf, work)

    tp = tvals * (lane < k).astype(jnp.float32)
    w = tp / (jnp.sum(tp, axis=-1, keepdims=True) + 1e-8)

    wts_ref[...] = w
    idx_ref[...] = tinds
    mask_ref[...] = mask
    k_ref[...] = jnp.broadcast_to(k, k_ref.shape)


def kernel(x, W, w1, b1, w2, b2):
    B, S, D = x.shape
    E = W.shape[1]
    N = B * S
    x_flat = x.reshape(N, D)
    wcat = jnp.concatenate(
        [w1, jnp.pad(W, ((0, 0), (0, 128 - E)))], axis=1
    ).astype(jnp.bfloat16)
    b1r = b1.reshape(1, -1)
    w2t = jnp.pad(w2, ((0, 0), (0, 127))).astype(jnp.bfloat16)
    b2r = b2.reshape(1, 1)

    grid = (N // _N_TILE,)
    T = _N_TILE
    wts, idx, mask, kout = pl.pallas_call(
        _router_body,
        grid=grid,
        in_specs=[
            pl.BlockSpec((T, D), lambda i: (i, 0)),
            pl.BlockSpec((D, _H + 128), lambda i: (0, 0)),
            pl.BlockSpec((1, _H), lambda i: (0, 0)),
            pl.BlockSpec((_H, 128), lambda i: (0, 0)),
            pl.BlockSpec((1, 1), lambda i: (0, 0)),
        ],
        out_specs=[
            pl.BlockSpec((T, E), lambda i: (i, 0)),
            pl.BlockSpec((T, E), lambda i: (i, 0)),
            pl.BlockSpec((T, E), lambda i: (i, 0)),
            pl.BlockSpec((T, E), lambda i: (i, 0)),
        ],
        out_shape=[
            jax.ShapeDtypeStruct((N, E), jnp.float32),
            jax.ShapeDtypeStruct((N, E), jnp.int32),
            jax.ShapeDtypeStruct((N, E), jnp.float32),
            jax.ShapeDtypeStruct((N, E), jnp.int32),
        ],
        compiler_params=pltpu.CompilerParams(
            dimension_semantics=("arbitrary",),
        ),
    )(x_flat, wcat, b1r, w2t, b2r)

    return (wts[:, :_MAX_K], idx[:, :_MAX_K],
            mask.reshape(B, S, E), kout[:, 0])
